# Initial kernel scaffold; baseline (speedup 1.0000x reference)
import jax
import jax.numpy as jnp
from jax.experimental import pallas as pl


def kernel(pc1, pc2, flow):
    raise NotImplementedError("write your pallas kernel here")


# speedup vs baseline: 3.2055x; 3.2055x over previous
import jax
import jax.numpy as jnp
from jax.experimental import pallas as pl
from jax.experimental.pallas import tpu as pltpu

_TM = 256


def _chamfer_body(pc1_ref, flow_ref, pc2t_ref, out_ref):
    n = pc1_ref.shape[1]
    p1 = pc1_ref[0] + flow_ref[0]
    c2x = pc2t_ref[0, 0:1, :]
    c2y = pc2t_ref[0, 1:2, :]
    c2z = pc2t_ref[0, 2:3, :]

    row_sum = jnp.zeros((), dtype=jnp.float32)
    col_min = jnp.full((1, n), jnp.inf, dtype=jnp.float32)
    for i in range(n // _TM):
        sl = p1[i * _TM:(i + 1) * _TM]
        dx = sl[:, 0:1] - c2x
        d = dx * dx
        dy = sl[:, 1:2] - c2y
        d += dy * dy
        dz = sl[:, 2:3] - c2z
        d += dz * dz
        row_min = jnp.min(d, axis=1)
        row_sum += jnp.sum(jnp.sqrt(row_min))
        col_min = jnp.minimum(col_min, jnp.min(d, axis=0, keepdims=True))

    total = row_sum + jnp.sum(jnp.sqrt(col_min))
    out_ref[0] = total.reshape(1, 1)


def kernel(pc1, pc2, flow):
    b, n, _ = pc1.shape
    pc2t = jnp.transpose(pc2, (0, 2, 1))
    partial = pl.pallas_call(
        _chamfer_body,
        grid=(b,),
        in_specs=[
            pl.BlockSpec((1, n, 3), lambda i: (i, 0, 0)),
            pl.BlockSpec((1, n, 3), lambda i: (i, 0, 0)),
            pl.BlockSpec((1, 3, n), lambda i: (i, 0, 0)),
        ],
        out_specs=pl.BlockSpec((1, 1, 1), lambda i: (i, 0, 0)),
        out_shape=jax.ShapeDtypeStruct((b, 1, 1), jnp.float32),
        compiler_params=pltpu.CompilerParams(
            dimension_semantics=("parallel",),
        ),
    )(pc1, flow, pc2t)
    return jnp.sum(partial) / (b * n)
